# baseline (speedup 1.0000x reference)
import functools

import jax
import jax.numpy as jnp
import numpy as np
from jax import lax
from jax.experimental import pallas as pl
from jax.experimental.pallas import tpu as pltpu
from jax.experimental.pallas import tpu_sc as plsc

VOCAB_SIZE = 100000
EMBED_DIM = 128
BATCH = 4096
SEQ = 50
NPAIR = SEQ // 2

NUM_CORES = 2
NUM_SUBCORES = 16
NUM_WORKERS = NUM_CORES * NUM_SUBCORES
BATCH_PER_WORKER = BATCH // NUM_WORKERS
SLOT = BATCH_PER_WORKER
NBUF = 3

SCALE = float(np.float32(np.sqrt(np.float32(EMBED_DIM))))

_mesh = plsc.VectorSubcoreMesh(core_axis_name="c", subcore_axis_name="s")


@functools.partial(
    pl.kernel,
    mesh=_mesh,
    out_type=jax.ShapeDtypeStruct((SEQ, BATCH, EMBED_DIM), jnp.float32),
    scratch_types=[
        pltpu.VMEM((SEQ, BATCH_PER_WORKER), jnp.int32),
        pltpu.VMEM((NBUF, 2, SLOT, EMBED_DIM), jnp.float32),
        [pltpu.SemaphoreType.DMA] * (2 * NBUF),
        [pltpu.SemaphoreType.DMA] * NBUF,
    ],
)
def _embed_lookup(x_hbm, tab_hbm, out_hbm, idx_v, rows_v, gsems, ssems):
    wid = lax.axis_index("s") * NUM_CORES + lax.axis_index("c")
    b0 = wid * BATCH_PER_WORKER

    pltpu.sync_copy(x_hbm.at[wid], idx_v)

    def gather_refs(p, slot, h):
        return tab_hbm.at[idx_v.at[2 * p + h]], rows_v.at[slot, h]

    def start_gathers(p, slot):
        for h in range(2):
            src, dst = gather_refs(p, slot, h)
            pltpu.async_copy(src, dst, gsems[2 * slot + h])

    def wait_gather(p, slot, h):
        src, dst = gather_refs(p, slot, h)
        pltpu.make_async_copy(src, dst, gsems[2 * slot + h]).wait()

    def store_refs(p, slot):
        return (
            rows_v.at[slot],
            out_hbm.at[pl.ds(2 * p, 2), pl.ds(b0, BATCH_PER_WORKER)],
        )

    def start_store(p, slot):
        src, dst = store_refs(p, slot)
        pltpu.async_copy(src, dst, ssems[slot])

    def wait_store(p, slot):
        src, dst = store_refs(p, slot)
        pltpu.make_async_copy(src, dst, ssems[slot]).wait()

    def scale_half(slot, h):
        def row_body(r, _):
            for j in range(EMBED_DIM // 16):
                sl = pl.ds(j * 16, 16)
                rows_v[slot, h, r, sl] = rows_v[slot, h, r, sl] * SCALE
            return _
        lax.fori_loop(0, SLOT, row_body, None, unroll=2)

    def step(p, slot):
        wait_gather(p, slot, 0)
        scale_half(slot, 0)
        wait_gather(p, slot, 1)
        scale_half(slot, 1)
        start_store(p, slot)

    start_gathers(0, 0)
    start_gathers(1, 1)
    step(0, 0)
    start_gathers(2, 2)
    step(1, 1)
    wait_store(0, 0)
    start_gathers(3, 0)

    def body(i, _):
        base = 2 + i * 3
        for s in range(NBUF):
            p = base + s
            slot = (2 + s) % NBUF
            step(p, slot)
            drain = (slot + 2) % NBUF
            wait_store(p - 1, drain)
            start_gathers(p + 2, drain)
        return _

    lax.fori_loop(0, 7, body, None)

    step(23, 23 % NBUF)
    wait_store(22, 1)
    step(24, 24 % NBUF)
    wait_store(23, 2)
    wait_store(24, 0)


def kernel(x, input_embedding):
    xprep = x.reshape(NUM_WORKERS, BATCH_PER_WORKER, SEQ).transpose(0, 2, 1)
    out_sm = _embed_lookup(xprep, input_embedding)
    return out_sm.transpose(1, 0, 2)


# speedup vs baseline: 1.0160x; 1.0160x over previous
import functools

import jax
import jax.numpy as jnp
import numpy as np
from jax import lax
from jax.experimental import pallas as pl
from jax.experimental.pallas import tpu as pltpu
from jax.experimental.pallas import tpu_sc as plsc

VOCAB_SIZE = 100000
EMBED_DIM = 128
BATCH = 4096
SEQ = 50

NUM_CORES = 2
NUM_SUBCORES = 16
NUM_WORKERS = NUM_CORES * NUM_SUBCORES
BATCH_PER_WORKER = BATCH // NUM_WORKERS
SLOT = BATCH_PER_WORKER
NBUF = 6

SCALE = float(np.float32(np.sqrt(np.float32(EMBED_DIM))))

_mesh = plsc.VectorSubcoreMesh(core_axis_name="c", subcore_axis_name="s")


@functools.partial(
    pl.kernel,
    mesh=_mesh,
    out_type=jax.ShapeDtypeStruct((SEQ, BATCH, EMBED_DIM), jnp.float32),
    scratch_types=[
        pltpu.VMEM((SEQ, BATCH_PER_WORKER), jnp.int32),
        pltpu.VMEM((NBUF * SLOT, EMBED_DIM), jnp.float32),
        [pltpu.SemaphoreType.DMA] * NBUF,
        [pltpu.SemaphoreType.DMA] * NBUF,
    ],
)
def _embed_lookup(x_hbm, tab_hbm, out_hbm, idx_v, rows_v, gsems, ssems):
    wid = lax.axis_index("s") * NUM_CORES + lax.axis_index("c")
    b0 = wid * BATCH_PER_WORKER

    pltpu.sync_copy(x_hbm.at[wid], idx_v)

    def gather_refs(si, slot):
        return tab_hbm.at[idx_v.at[si]], rows_v.at[pl.ds(slot * SLOT, SLOT)]

    def start_gather(si, slot):
        src, dst = gather_refs(si, slot)
        pltpu.async_copy(src, dst, gsems[slot])

    def wait_gather(si, slot):
        src, dst = gather_refs(si, slot)
        pltpu.make_async_copy(src, dst, gsems[slot]).wait()

    def store_refs(si, slot):
        return (
            rows_v.at[pl.ds(slot * SLOT, SLOT)],
            out_hbm.at[si, pl.ds(b0, BATCH_PER_WORKER)],
        )

    def start_store(si, slot):
        src, dst = store_refs(si, slot)
        pltpu.async_copy(src, dst, ssems[slot])

    def wait_store(si, slot):
        src, dst = store_refs(si, slot)
        pltpu.make_async_copy(src, dst, ssems[slot]).wait()

    def scale_slot(slot):
        def row_body(r, _):
            for j in range(EMBED_DIM // 16):
                sl = pl.ds(j * 16, 16)
                rows_v[slot * SLOT + r, sl] = rows_v[slot * SLOT + r, sl] * SCALE
            return _
        lax.fori_loop(0, SLOT, row_body, None, unroll=2)

    def step(si, slot):
        wait_gather(si, slot)
        scale_slot(slot)
        start_store(si, slot)

    for p in range(4):
        start_gather(p, p)
    step(0, 0)
    start_gather(4, 4)
    step(1, 1)
    start_gather(5, 5)

    def body(i, _):
        base = 2 + i * 6
        for s in range(NBUF):
            si = base + s
            slot = (2 + s) % NBUF
            wait_gather(si, slot)
            scale_slot(slot)
            start_store(si, slot)
            drain = (slot + 4) % NBUF
            wait_store(si - 2, drain)
            start_gather(si + 4, drain)
        return _

    lax.fori_loop(0, (SEQ - 8) // NBUF, body, None)

    step(44, 44 % NBUF)
    wait_store(42, 0)
    start_gather(48, 0)
    step(45, 45 % NBUF)
    wait_store(43, 1)
    start_gather(49, 1)
    for si in range(46, 50):
        step(si, si % NBUF)
        wait_store(si - 2, (si - 2) % NBUF)
    wait_store(48, 0)
    wait_store(49, 1)


def kernel(x, input_embedding):
    xprep = x.reshape(NUM_WORKERS, BATCH_PER_WORKER, SEQ).transpose(0, 2, 1)
    out_sm = _embed_lookup(xprep, input_embedding)
    return out_sm.transpose(1, 0, 2)
